# Initial kernel scaffold; baseline (speedup 1.0000x reference)
import jax
import jax.numpy as jnp
from jax.experimental import pallas as pl


def kernel(x, edge_index, edge_attr, edge_W1, edge_b1, edge_W2, edge_b2, node_W1, node_b1, node_W2, node_b2):
    raise NotImplementedError("write your pallas kernel here")


# speedup vs baseline: 1.8106x; 1.8106x over previous
import functools

import jax
import jax.numpy as jnp
from jax import lax
from jax.experimental import pallas as pl
from jax.experimental.pallas import tpu as pltpu
from jax.experimental.pallas import tpu_sc as plsc

_N = 10000
_E = 160000
_D = 256
_STEPS = 4

_NC = 2
_NS = 16
_NW = _NC * _NS

_CH = 128
_E_PAD = 163840
_N_PAD = 10240
_GCH = _E_PAD // _NW // _CH
_SCH = _E_PAD // _NS // _CH
_ROWS_T = _N_PAD // _NS
_HALF = _D // _NC

_BLK_E = 2048
_BLK_N = 1024


def _proj_body(x_ref, wa_ref, wb_ref, a_ref, b_ref):
    xv = x_ref[...]
    a_ref[...] = jnp.dot(xv, wa_ref[...], preferred_element_type=jnp.float32)
    b_ref[...] = jnp.dot(xv, wb_ref[...], preferred_element_type=jnp.float32)


def _edge_body(gs_ref, gd_ref, e_ref, wc_ref, b1_ref, w2_ref, b2_ref, out_ref):
    ev = e_ref[...]
    h = gs_ref[...] + gd_ref[...] + b1_ref[...]
    h = h + jnp.dot(ev, wc_ref[...], preferred_element_type=jnp.float32)
    h = jnp.maximum(h, 0.0)
    out_ref[...] = jnp.dot(h, w2_ref[...], preferred_element_type=jnp.float32) + b2_ref[...] + ev


def _node_body(x_ref, g_ref, wa_ref, wb_ref, b1_ref, w2_ref, b2_ref, out_ref):
    xv = x_ref[...]
    h = jnp.dot(xv, wa_ref[...], preferred_element_type=jnp.float32)
    h = h + jnp.dot(g_ref[...], wb_ref[...], preferred_element_type=jnp.float32)
    h = jnp.maximum(h + b1_ref[...], 0.0)
    out_ref[...] = jnp.dot(h, w2_ref[...], preferred_element_type=jnp.float32) + b2_ref[...] + xv


def _row_spec(blk):
    return pl.BlockSpec((blk, _D), lambda i: (i, 0))


def _full_spec(shape):
    return pl.BlockSpec(shape, lambda i: (0,) * len(shape))


_proj = pl.pallas_call(
    _proj_body,
    grid=(_N_PAD // _BLK_N,),
    in_specs=[_row_spec(_BLK_N), _full_spec((_D, _D)), _full_spec((_D, _D))],
    out_specs=[_row_spec(_BLK_N), _row_spec(_BLK_N)],
    out_shape=[jax.ShapeDtypeStruct((_N_PAD, _D), jnp.float32)] * 2,
    compiler_params=pltpu.CompilerParams(dimension_semantics=("parallel",)),
)

_edge_mlp = pl.pallas_call(
    _edge_body,
    grid=(_E_PAD // _BLK_E,),
    in_specs=[_row_spec(_BLK_E), _row_spec(_BLK_E), _row_spec(_BLK_E),
              _full_spec((_D, _D)), _full_spec((1, _D)),
              _full_spec((_D, _D)), _full_spec((1, _D))],
    out_specs=_row_spec(_BLK_E),
    out_shape=jax.ShapeDtypeStruct((_E_PAD, _D), jnp.float32),
    compiler_params=pltpu.CompilerParams(dimension_semantics=("parallel",)),
)

_node_mlp = pl.pallas_call(
    _node_body,
    grid=(_N_PAD // _BLK_N,),
    in_specs=[_row_spec(_BLK_N), _row_spec(_BLK_N),
              _full_spec((_D, _D)), _full_spec((_D, _D)), _full_spec((1, _D)),
              _full_spec((_D, _D)), _full_spec((1, _D))],
    out_specs=_row_spec(_BLK_N),
    out_shape=jax.ShapeDtypeStruct((_N_PAD, _D), jnp.float32),
    compiler_params=pltpu.CompilerParams(dimension_semantics=("parallel",)),
)


_SC_MESH = plsc.VectorSubcoreMesh(
    core_axis_name="c", subcore_axis_name="s", num_cores=_NC, num_subcores=_NS)


@functools.partial(
    pl.kernel,
    out_type=(jax.ShapeDtypeStruct((_E_PAD, _D), jnp.float32),
              jax.ShapeDtypeStruct((_E_PAD, _D), jnp.float32)),
    mesh=_SC_MESH,
    scratch_types=[
        pltpu.VMEM((_GCH, _CH), jnp.int32),
        pltpu.VMEM((_GCH, _CH), jnp.int32),
        pltpu.VMEM((_CH, _D), jnp.float32),
        pltpu.VMEM((_CH, _D), jnp.float32),
        pltpu.SemaphoreType.DMA,
        pltpu.SemaphoreType.DMA,
    ],
)
def _sc_gather(a_hbm, b_hbm, src_hbm, dst_hbm, gs_hbm, gd_hbm,
               srcv, dstv, bufa, bufb, sema, semb):
    wid = lax.axis_index("s") * _NC + lax.axis_index("c")
    row0 = wid * _GCH
    pltpu.sync_copy(src_hbm.at[pl.ds(row0, _GCH)], srcv)
    pltpu.sync_copy(dst_hbm.at[pl.ds(row0, _GCH)], dstv)

    def body(j, carry):
        ca = pltpu.async_copy(a_hbm.at[srcv.at[j]], bufa, sema)
        cb = pltpu.async_copy(b_hbm.at[dstv.at[j]], bufb, semb)
        ca.wait()
        cb.wait()
        out0 = (row0 + j) * _CH
        pltpu.sync_copy(bufa, gs_hbm.at[pl.ds(out0, _CH)])
        pltpu.sync_copy(bufb, gd_hbm.at[pl.ds(out0, _CH)])
        return carry

    lax.fori_loop(0, _GCH, body, 0)


@functools.partial(
    pl.kernel,
    out_type=jax.ShapeDtypeStruct((_N_PAD, _D), jnp.float32),
    mesh=_SC_MESH,
    scratch_types=[
        pltpu.VMEM((_SCH, _CH), jnp.int32),
        pltpu.VMEM((_CH, _HALF), jnp.float32),
        pltpu.VMEM_SHARED((_N_PAD, _HALF), jnp.float32),
    ],
)
def _sc_scatter(e_hbm, dst_hbm, agg_hbm, dstv, ebuf, acc):
    c = lax.axis_index("c")
    s = lax.axis_index("s")
    col0 = c * _HALF

    def zbody(r, carry):
        zero = jnp.zeros((16,), jnp.float32)
        for k in range(_HALF // 16):
            ebuf[r, pl.ds(k * 16, 16)] = zero
        return carry

    lax.fori_loop(0, _CH, zbody, 0)
    for k in range(_ROWS_T // _CH):
        pltpu.sync_copy(ebuf, acc.at[pl.ds(s * _ROWS_T + k * _CH, _CH)])
    plsc.subcore_barrier()

    row0 = s * _SCH

    def body(j, carry):
        e0 = (row0 + j) * _CH
        pltpu.sync_copy(e_hbm.at[pl.ds(e0, _CH), pl.ds(col0, _HALF)], ebuf)
        pltpu.sync_copy(ebuf, acc.at[dstv.at[j]], add=True)
        return carry

    pltpu.sync_copy(dst_hbm.at[pl.ds(row0, _SCH)], dstv)
    lax.fori_loop(0, _SCH, body, 0)
    plsc.subcore_barrier()

    pltpu.sync_copy(acc.at[pl.ds(s * _ROWS_T, _ROWS_T)],
                    agg_hbm.at[pl.ds(s * _ROWS_T, _ROWS_T), pl.ds(col0, _HALF)])


def kernel(x, edge_index, edge_attr, edge_W1, edge_b1, edge_W2, edge_b2,
           node_W1, node_b1, node_W2, node_b2):
    src = edge_index[0]
    dst = edge_index[1]
    pad_e = _E_PAD - _E
    src_p = jnp.concatenate([src, jnp.zeros((pad_e,), jnp.int32)]).reshape(-1, _CH)
    dst_p = jnp.concatenate([dst, jnp.full((pad_e,), _N, jnp.int32)]).reshape(-1, _CH)
    x_c = jnp.pad(x, ((0, _N_PAD - _N), (0, 0)))
    e_c = jnp.pad(edge_attr, ((0, pad_e), (0, 0)))

    for step in range(_STEPS):
        wa = edge_W1[step, :_D]
        wb = edge_W1[step, _D:2 * _D]
        wc = edge_W1[step, 2 * _D:]
        eb1 = edge_b1[step].reshape(1, _D)
        ew2 = edge_W2[step]
        eb2 = edge_b2[step].reshape(1, _D)
        na = node_W1[step, :_D]
        nb = node_W1[step, _D:]
        nb1 = node_b1[step].reshape(1, _D)
        nw2 = node_W2[step]
        nb2 = node_b2[step].reshape(1, _D)

        a_t, b_t = _proj(x_c, wa, wb)
        gs, gd = _sc_gather(a_t, b_t, src_p, dst_p)
        e_c = _edge_mlp(gs, gd, e_c, wc, eb1, ew2, eb2)
        agg = _sc_scatter(e_c, dst_p)
        x_c = _node_mlp(x_c, agg, na, nb, nb1, nw2, nb2)

    return x_c[:_N], e_c[:_E]


# speedup vs baseline: 2.0967x; 1.1580x over previous
import functools

import jax
import jax.numpy as jnp
from jax import lax
from jax.experimental import pallas as pl
from jax.experimental.pallas import tpu as pltpu
from jax.experimental.pallas import tpu_sc as plsc

_N = 10000
_E = 160000
_D = 256
_STEPS = 4

_NC = 2
_NS = 16
_NW = _NC * _NS

_CH = 128
_CHG = 80
_E_PAD = 163840
_N_PAD = 10240
_GCH = _E_PAD // _NW // _CHG
_SCH = _E_PAD // _NS // _CH
_ROWS_T = _N_PAD // _NS
_HALF = _D // _NC

_BLK_E = 2048
_BLK_N = 1024


def _proj_body(x_ref, wa_ref, wb_ref, a_ref, b_ref):
    xv = x_ref[...]
    a_ref[...] = jnp.dot(xv, wa_ref[...], preferred_element_type=jnp.float32)
    b_ref[...] = jnp.dot(xv, wb_ref[...], preferred_element_type=jnp.float32)


def _edge_body(gs_ref, gd_ref, e_ref, wc_ref, b1_ref, w2_ref, b2_ref, out_ref):
    ev = e_ref[...]
    h = gs_ref[...] + gd_ref[...] + b1_ref[...]
    h = h + jnp.dot(ev, wc_ref[...], preferred_element_type=jnp.float32)
    h = jnp.maximum(h, 0.0)
    out_ref[...] = jnp.dot(h, w2_ref[...], preferred_element_type=jnp.float32) + b2_ref[...] + ev


def _node_body(x_ref, g_ref, wa_ref, wb_ref, b1_ref, w2_ref, b2_ref, out_ref):
    xv = x_ref[...]
    h = jnp.dot(xv, wa_ref[...], preferred_element_type=jnp.float32)
    h = h + jnp.dot(g_ref[...], wb_ref[...], preferred_element_type=jnp.float32)
    h = jnp.maximum(h + b1_ref[...], 0.0)
    out_ref[...] = jnp.dot(h, w2_ref[...], preferred_element_type=jnp.float32) + b2_ref[...] + xv


def _row_spec(blk):
    return pl.BlockSpec((blk, _D), lambda i: (i, 0))


def _full_spec(shape):
    return pl.BlockSpec(shape, lambda i: (0,) * len(shape))


_proj = pl.pallas_call(
    _proj_body,
    grid=(_N_PAD // _BLK_N,),
    in_specs=[_row_spec(_BLK_N), _full_spec((_D, _D)), _full_spec((_D, _D))],
    out_specs=[_row_spec(_BLK_N), _row_spec(_BLK_N)],
    out_shape=[jax.ShapeDtypeStruct((_N_PAD, _D), jnp.float32)] * 2,
    compiler_params=pltpu.CompilerParams(dimension_semantics=("parallel",)),
)

_edge_mlp = pl.pallas_call(
    _edge_body,
    grid=(_E_PAD // _BLK_E,),
    in_specs=[_row_spec(_BLK_E), _row_spec(_BLK_E), _row_spec(_BLK_E),
              _full_spec((_D, _D)), _full_spec((1, _D)),
              _full_spec((_D, _D)), _full_spec((1, _D))],
    out_specs=_row_spec(_BLK_E),
    out_shape=jax.ShapeDtypeStruct((_E_PAD, _D), jnp.float32),
    compiler_params=pltpu.CompilerParams(dimension_semantics=("parallel",)),
)

_node_mlp = pl.pallas_call(
    _node_body,
    grid=(_N_PAD // _BLK_N,),
    in_specs=[_row_spec(_BLK_N), _row_spec(_BLK_N),
              _full_spec((_D, _D)), _full_spec((_D, _D)), _full_spec((1, _D)),
              _full_spec((_D, _D)), _full_spec((1, _D))],
    out_specs=_row_spec(_BLK_N),
    out_shape=jax.ShapeDtypeStruct((_N_PAD, _D), jnp.float32),
    compiler_params=pltpu.CompilerParams(dimension_semantics=("parallel",)),
)


_SC_MESH = plsc.VectorSubcoreMesh(
    core_axis_name="c", subcore_axis_name="s", num_cores=_NC, num_subcores=_NS)


@functools.partial(
    pl.kernel,
    out_type=(jax.ShapeDtypeStruct((_E_PAD, _D), jnp.float32),
              jax.ShapeDtypeStruct((_E_PAD, _D), jnp.float32)),
    mesh=_SC_MESH,
    scratch_types=[
        pltpu.VMEM((_GCH, _CHG), jnp.int32),
        pltpu.VMEM((_GCH, _CHG), jnp.int32),
        pltpu.VMEM((_CHG, _D), jnp.float32),
        pltpu.VMEM((_CHG, _D), jnp.float32),
        pltpu.VMEM((_CHG, _D), jnp.float32),
        pltpu.VMEM((_CHG, _D), jnp.float32),
        pltpu.SemaphoreType.DMA,
        pltpu.SemaphoreType.DMA,
        pltpu.SemaphoreType.DMA,
        pltpu.SemaphoreType.DMA,
    ],
)
def _sc_gather(a_hbm, b_hbm, src_hbm, dst_hbm, gs_hbm, gd_hbm,
               srcv, dstv, bufa0, bufb0, bufa1, bufb1, sa0, sb0, sa1, sb1):
    wid = lax.axis_index("s") * _NC + lax.axis_index("c")
    row0 = wid * _GCH
    pltpu.sync_copy(src_hbm.at[pl.ds(row0, _GCH)], srcv)
    pltpu.sync_copy(dst_hbm.at[pl.ds(row0, _GCH)], dstv)

    def start(j, ba, bb, sa, sb):
        pltpu.async_copy(a_hbm.at[srcv.at[j]], ba, sa)
        pltpu.async_copy(b_hbm.at[dstv.at[j]], bb, sb)

    def finish(j, ba, bb, sa, sb):
        pltpu.make_async_copy(a_hbm.at[srcv.at[0]], ba, sa).wait()
        pltpu.make_async_copy(b_hbm.at[dstv.at[0]], bb, sb).wait()
        out0 = (row0 + j) * _CHG
        pltpu.sync_copy(ba, gs_hbm.at[pl.ds(out0, _CHG)])
        pltpu.sync_copy(bb, gd_hbm.at[pl.ds(out0, _CHG)])

    start(0, bufa0, bufb0, sa0, sb0)

    def body(t, carry):
        j = 2 * t
        start(j + 1, bufa1, bufb1, sa1, sb1)
        finish(j, bufa0, bufb0, sa0, sb0)
        start(j + 2, bufa0, bufb0, sa0, sb0)
        finish(j + 1, bufa1, bufb1, sa1, sb1)
        return carry

    lax.fori_loop(0, _GCH // 2 - 1, body, 0)
    start(_GCH - 1, bufa1, bufb1, sa1, sb1)
    finish(_GCH - 2, bufa0, bufb0, sa0, sb0)
    finish(_GCH - 1, bufa1, bufb1, sa1, sb1)


@functools.partial(
    pl.kernel,
    out_type=jax.ShapeDtypeStruct((_N_PAD, _D), jnp.float32),
    mesh=_SC_MESH,
    scratch_types=[
        pltpu.VMEM((_SCH, _CH), jnp.int32),
        pltpu.VMEM((_CH, _HALF), jnp.float32),
        pltpu.VMEM((_CH, _HALF), jnp.float32),
        pltpu.VMEM_SHARED((_N_PAD, _HALF), jnp.float32),
        pltpu.SemaphoreType.DMA,
        pltpu.SemaphoreType.DMA,
    ],
)
def _sc_scatter(e_hbm, dst_hbm, agg_hbm, dstv, ebuf, ebuf1, acc, se0, se1):
    c = lax.axis_index("c")
    s = lax.axis_index("s")
    col0 = c * _HALF

    def zbody(r, carry):
        zero = jnp.zeros((16,), jnp.float32)
        for k in range(_HALF // 16):
            ebuf[r, pl.ds(k * 16, 16)] = zero
        return carry

    lax.fori_loop(0, _CH, zbody, 0)
    for k in range(_ROWS_T // _CH):
        pltpu.sync_copy(ebuf, acc.at[pl.ds(s * _ROWS_T + k * _CH, _CH)])
    plsc.subcore_barrier()

    row0 = s * _SCH
    pltpu.sync_copy(dst_hbm.at[pl.ds(row0, _SCH)], dstv)

    def start(j, buf, sem):
        e0 = (row0 + j) * _CH
        pltpu.async_copy(e_hbm.at[pl.ds(e0, _CH), pl.ds(col0, _HALF)], buf, sem)

    def finish(j, buf, sem):
        pltpu.make_async_copy(e_hbm.at[pl.ds(0, _CH), pl.ds(col0, _HALF)], buf, sem).wait()
        pltpu.sync_copy(buf, acc.at[dstv.at[j]], add=True)

    start(0, ebuf, se0)

    def body(t, carry):
        j = 2 * t
        start(j + 1, ebuf1, se1)
        finish(j, ebuf, se0)
        start(j + 2, ebuf, se0)
        finish(j + 1, ebuf1, se1)
        return carry

    lax.fori_loop(0, _SCH // 2 - 1, body, 0)
    start(_SCH - 1, ebuf1, se1)
    finish(_SCH - 2, ebuf, se0)
    finish(_SCH - 1, ebuf1, se1)
    plsc.subcore_barrier()

    pltpu.sync_copy(acc.at[pl.ds(s * _ROWS_T, _ROWS_T)],
                    agg_hbm.at[pl.ds(s * _ROWS_T, _ROWS_T), pl.ds(col0, _HALF)])


def kernel(x, edge_index, edge_attr, edge_W1, edge_b1, edge_W2, edge_b2,
           node_W1, node_b1, node_W2, node_b2):
    src = edge_index[0]
    dst = edge_index[1]
    pad_e = _E_PAD - _E
    src_flat = jnp.concatenate([src, jnp.zeros((pad_e,), jnp.int32)])
    dst_flat = jnp.concatenate([dst, jnp.full((pad_e,), _N, jnp.int32)])
    src_g = src_flat.reshape(-1, _CHG)
    dst_g = dst_flat.reshape(-1, _CHG)
    dst_s = dst_flat.reshape(-1, _CH)
    x_c = jnp.pad(x, ((0, _N_PAD - _N), (0, 0)))
    e_c = jnp.pad(edge_attr, ((0, pad_e), (0, 0)))

    for step in range(_STEPS):
        wa = edge_W1[step, :_D]
        wb = edge_W1[step, _D:2 * _D]
        wc = edge_W1[step, 2 * _D:]
        eb1 = edge_b1[step].reshape(1, _D)
        ew2 = edge_W2[step]
        eb2 = edge_b2[step].reshape(1, _D)
        na = node_W1[step, :_D]
        nb = node_W1[step, _D:]
        nb1 = node_b1[step].reshape(1, _D)
        nw2 = node_W2[step]
        nb2 = node_b2[step].reshape(1, _D)

        a_t, b_t = _proj(x_c, wa, wb)
        gs, gd = _sc_gather(a_t, b_t, src_g, dst_g)
        e_c = _edge_mlp(gs, gd, e_c, wc, eb1, ew2, eb2)
        agg = _sc_scatter(e_c, dst_s)
        x_c = _node_mlp(x_c, agg, na, nb, nb1, nw2, nb2)

    return x_c[:_N], e_c[:_E]


# speedup vs baseline: 2.7331x; 1.3035x over previous
import functools

import jax
import jax.numpy as jnp
from jax import lax
from jax.experimental import pallas as pl
from jax.experimental.pallas import tpu as pltpu
from jax.experimental.pallas import tpu_sc as plsc

_N = 10000
_E = 160000
_D = 256
_STEPS = 4

_NC = 2
_NS = 16
_NW = _NC * _NS

_CH = 128
_E_PAD = 163840
_N_PAD = 10240
_GCH = _E_PAD // _NW // _CH
_SCH = _E_PAD // _NS // _CH
_ROWS_T = _N_PAD // _NS
_HALF = _D // _NC

_BLK_E = 2048
_BLK_N = 1024


def _pack_bf16_pair(val):
    def rnd(b):
        return (b + 0x7FFF + ((b >> 16) & 1)) >> 16

    lo = rnd(jax.lax.bitcast_convert_type(val[:, :_HALF], jnp.int32)) & 0xFFFF
    hi = rnd(jax.lax.bitcast_convert_type(val[:, _HALF:], jnp.int32)) << 16
    return lo | hi


def _unpack_bf16_pair(g32):
    lo = jax.lax.bitcast_convert_type(g32 << 16, jnp.float32)
    hi = jax.lax.bitcast_convert_type(g32 & jnp.int32(-65536), jnp.float32)
    return lo, hi


def _proj_body(x_ref, wa_ref, wb_ref, a_ref, b_ref):
    xv = x_ref[...]
    a_ref[...] = _pack_bf16_pair(jnp.dot(xv, wa_ref[...], preferred_element_type=jnp.float32))
    b_ref[...] = _pack_bf16_pair(jnp.dot(xv, wb_ref[...], preferred_element_type=jnp.float32))


def _edge_body(gs_ref, gd_ref, e_ref, wc_ref, b1_ref, w2_ref, b2_ref, out_ref):
    ev = e_ref[...]
    gs_lo, gs_hi = _unpack_bf16_pair(gs_ref[...])
    gd_lo, gd_hi = _unpack_bf16_pair(gd_ref[...])
    h = jnp.concatenate([gs_lo + gd_lo, gs_hi + gd_hi], axis=-1) + b1_ref[...]
    h = h + jnp.dot(ev, wc_ref[...], preferred_element_type=jnp.float32)
    h = jnp.maximum(h, 0.0)
    out_ref[...] = jnp.dot(h, w2_ref[...], preferred_element_type=jnp.float32) + b2_ref[...] + ev


def _node_body(x_ref, g_ref, wa_ref, wb_ref, b1_ref, w2_ref, b2_ref, out_ref):
    xv = x_ref[...]
    h = jnp.dot(xv, wa_ref[...], preferred_element_type=jnp.float32)
    h = h + jnp.dot(g_ref[...], wb_ref[...], preferred_element_type=jnp.float32)
    h = jnp.maximum(h + b1_ref[...], 0.0)
    out_ref[...] = jnp.dot(h, w2_ref[...], preferred_element_type=jnp.float32) + b2_ref[...] + xv


def _row_spec(blk):
    return pl.BlockSpec((blk, _D), lambda i: (i, 0))


def _full_spec(shape):
    return pl.BlockSpec(shape, lambda i: (0,) * len(shape))


def _half_spec(blk):
    return pl.BlockSpec((blk, _HALF), lambda i: (i, 0))


_proj = pl.pallas_call(
    _proj_body,
    grid=(_N_PAD // _BLK_N,),
    in_specs=[_row_spec(_BLK_N), _full_spec((_D, _D)), _full_spec((_D, _D))],
    out_specs=[_half_spec(_BLK_N), _half_spec(_BLK_N)],
    out_shape=[jax.ShapeDtypeStruct((_N_PAD, _HALF), jnp.int32)] * 2,
    compiler_params=pltpu.CompilerParams(dimension_semantics=("parallel",)),
)

_edge_mlp = pl.pallas_call(
    _edge_body,
    grid=(_E_PAD // _BLK_E,),
    in_specs=[_half_spec(_BLK_E), _half_spec(_BLK_E), _row_spec(_BLK_E),
              _full_spec((_D, _D)), _full_spec((1, _D)),
              _full_spec((_D, _D)), _full_spec((1, _D))],
    out_specs=_row_spec(_BLK_E),
    out_shape=jax.ShapeDtypeStruct((_E_PAD, _D), jnp.float32),
    compiler_params=pltpu.CompilerParams(dimension_semantics=("parallel",)),
)

_node_mlp = pl.pallas_call(
    _node_body,
    grid=(_N_PAD // _BLK_N,),
    in_specs=[_row_spec(_BLK_N), _row_spec(_BLK_N),
              _full_spec((_D, _D)), _full_spec((_D, _D)), _full_spec((1, _D)),
              _full_spec((_D, _D)), _full_spec((1, _D))],
    out_specs=_row_spec(_BLK_N),
    out_shape=jax.ShapeDtypeStruct((_N_PAD, _D), jnp.float32),
    compiler_params=pltpu.CompilerParams(dimension_semantics=("parallel",)),
)


_SC_MESH = plsc.VectorSubcoreMesh(
    core_axis_name="c", subcore_axis_name="s", num_cores=_NC, num_subcores=_NS)


@functools.partial(
    pl.kernel,
    out_type=(jax.ShapeDtypeStruct((_E_PAD, _HALF), jnp.int32),
              jax.ShapeDtypeStruct((_E_PAD, _HALF), jnp.int32)),
    mesh=_SC_MESH,
    scratch_types=[
        pltpu.VMEM((_GCH, _CH), jnp.int32),
        pltpu.VMEM((_GCH, _CH), jnp.int32),
        pltpu.VMEM((_CH, _HALF), jnp.int32),
        pltpu.VMEM((_CH, _HALF), jnp.int32),
        pltpu.VMEM((_CH, _HALF), jnp.int32),
        pltpu.VMEM((_CH, _HALF), jnp.int32),
        pltpu.SemaphoreType.DMA,
        pltpu.SemaphoreType.DMA,
        pltpu.SemaphoreType.DMA,
        pltpu.SemaphoreType.DMA,
    ],
)
def _sc_gather(a_hbm, b_hbm, src_hbm, dst_hbm, gs_hbm, gd_hbm,
               srcv, dstv, bufa0, bufb0, bufa1, bufb1, sa0, sb0, sa1, sb1):
    wid = lax.axis_index("s") * _NC + lax.axis_index("c")
    row0 = wid * _GCH
    pltpu.sync_copy(src_hbm.at[pl.ds(row0, _GCH)], srcv)
    pltpu.sync_copy(dst_hbm.at[pl.ds(row0, _GCH)], dstv)

    def start(j, ba, bb, sa, sb):
        pltpu.async_copy(a_hbm.at[srcv.at[j]], ba, sa)
        pltpu.async_copy(b_hbm.at[dstv.at[j]], bb, sb)

    def finish(j, ba, bb, sa, sb):
        pltpu.make_async_copy(a_hbm.at[srcv.at[0]], ba, sa).wait()
        pltpu.make_async_copy(b_hbm.at[dstv.at[0]], bb, sb).wait()
        out0 = (row0 + j) * _CH
        pltpu.sync_copy(ba, gs_hbm.at[pl.ds(out0, _CH)])
        pltpu.sync_copy(bb, gd_hbm.at[pl.ds(out0, _CH)])

    start(0, bufa0, bufb0, sa0, sb0)

    def body(t, carry):
        j = 2 * t
        start(j + 1, bufa1, bufb1, sa1, sb1)
        finish(j, bufa0, bufb0, sa0, sb0)
        start(j + 2, bufa0, bufb0, sa0, sb0)
        finish(j + 1, bufa1, bufb1, sa1, sb1)
        return carry

    lax.fori_loop(0, _GCH // 2 - 1, body, 0)
    start(_GCH - 1, bufa1, bufb1, sa1, sb1)
    finish(_GCH - 2, bufa0, bufb0, sa0, sb0)
    finish(_GCH - 1, bufa1, bufb1, sa1, sb1)


@functools.partial(
    pl.kernel,
    out_type=jax.ShapeDtypeStruct((_N_PAD, _D), jnp.float32),
    mesh=_SC_MESH,
    scratch_types=[
        pltpu.VMEM((_SCH, _CH), jnp.int32),
        pltpu.VMEM((_CH, _HALF), jnp.float32),
        pltpu.VMEM((_CH, _HALF), jnp.float32),
        pltpu.VMEM_SHARED((_N_PAD, _HALF), jnp.float32),
        pltpu.SemaphoreType.DMA,
        pltpu.SemaphoreType.DMA,
    ],
)
def _sc_scatter(e_hbm, dst_hbm, agg_hbm, dstv, ebuf, ebuf1, acc, se0, se1):
    c = lax.axis_index("c")
    s = lax.axis_index("s")
    col0 = c * _HALF

    def zbody(r, carry):
        zero = jnp.zeros((16,), jnp.float32)
        for k in range(_HALF // 16):
            ebuf[r, pl.ds(k * 16, 16)] = zero
        return carry

    lax.fori_loop(0, _CH, zbody, 0)
    for k in range(_ROWS_T // _CH):
        pltpu.sync_copy(ebuf, acc.at[pl.ds(s * _ROWS_T + k * _CH, _CH)])
    plsc.subcore_barrier()

    row0 = s * _SCH
    pltpu.sync_copy(dst_hbm.at[pl.ds(row0, _SCH)], dstv)

    def start(j, buf, sem):
        e0 = (row0 + j) * _CH
        pltpu.async_copy(e_hbm.at[pl.ds(e0, _CH), pl.ds(col0, _HALF)], buf, sem)

    def finish(j, buf, sem):
        pltpu.make_async_copy(e_hbm.at[pl.ds(0, _CH), pl.ds(col0, _HALF)], buf, sem).wait()
        pltpu.sync_copy(buf, acc.at[dstv.at[j]], add=True)

    start(0, ebuf, se0)

    def body(t, carry):
        j = 2 * t
        start(j + 1, ebuf1, se1)
        finish(j, ebuf, se0)
        start(j + 2, ebuf, se0)
        finish(j + 1, ebuf1, se1)
        return carry

    lax.fori_loop(0, _SCH // 2 - 1, body, 0)
    start(_SCH - 1, ebuf1, se1)
    finish(_SCH - 2, ebuf, se0)
    finish(_SCH - 1, ebuf1, se1)
    plsc.subcore_barrier()

    pltpu.sync_copy(acc.at[pl.ds(s * _ROWS_T, _ROWS_T)],
                    agg_hbm.at[pl.ds(s * _ROWS_T, _ROWS_T), pl.ds(col0, _HALF)])


def kernel(x, edge_index, edge_attr, edge_W1, edge_b1, edge_W2, edge_b2,
           node_W1, node_b1, node_W2, node_b2):
    src = edge_index[0]
    dst = edge_index[1]
    pad_e = _E_PAD - _E
    src_flat = jnp.concatenate([src, jnp.zeros((pad_e,), jnp.int32)])
    dst_flat = jnp.concatenate([dst, jnp.full((pad_e,), _N, jnp.int32)])
    src_g = src_flat.reshape(-1, _CH)
    dst_s = dst_flat.reshape(-1, _CH)
    x_c = jnp.pad(x, ((0, _N_PAD - _N), (0, 0)))
    e_c = jnp.pad(edge_attr, ((0, pad_e), (0, 0)))

    for step in range(_STEPS):
        wa = edge_W1[step, :_D]
        wb = edge_W1[step, _D:2 * _D]
        wc = edge_W1[step, 2 * _D:]
        eb1 = edge_b1[step].reshape(1, _D)
        ew2 = edge_W2[step]
        eb2 = edge_b2[step].reshape(1, _D)
        na = node_W1[step, :_D]
        nb = node_W1[step, _D:]
        nb1 = node_b1[step].reshape(1, _D)
        nw2 = node_W2[step]
        nb2 = node_b2[step].reshape(1, _D)

        a_t, b_t = _proj(x_c, wa, wb)
        gs, gd = _sc_gather(a_t, b_t, src_g, dst_s)
        e_c = _edge_mlp(gs, gd, e_c, wc, eb1, ew2, eb2)
        agg = _sc_scatter(e_c, dst_s)
        x_c = _node_mlp(x_c, agg, na, nb, nb1, nw2, nb2)

    return x_c[:_N], e_c[:_E]


# speedup vs baseline: 2.7601x; 1.0099x over previous
import functools

import jax
import jax.numpy as jnp
from jax import lax
from jax.experimental import pallas as pl
from jax.experimental.pallas import tpu as pltpu
from jax.experimental.pallas import tpu_sc as plsc

_N = 10000
_E = 160000
_D = 256
_STEPS = 4

_NC = 2
_NS = 16
_NW = _NC * _NS

_CH = 128
_E_PAD = 163840
_EH = _E_PAD // 2
_N_PAD = 10240
_GCH = _EH // _NW // _CH
_SCH = _EH // _NS // _CH
_ROWS_T = _N_PAD // _NS
_HALF = _D // _NC

_BLK_E = 2048
_BLK_N = 1024


def _pack_bf16_pair(val):
    def rnd(b):
        return (b + 0x7FFF + ((b >> 16) & 1)) >> 16

    lo = rnd(jax.lax.bitcast_convert_type(val[:, :_HALF], jnp.int32)) & 0xFFFF
    hi = rnd(jax.lax.bitcast_convert_type(val[:, _HALF:], jnp.int32)) << 16
    return lo | hi


def _unpack_bf16_pair(g32):
    lo = jax.lax.bitcast_convert_type(g32 << 16, jnp.float32)
    hi = jax.lax.bitcast_convert_type(g32 & jnp.int32(-65536), jnp.float32)
    return lo, hi


def _proj_body(x_ref, wa_ref, wb_ref, a_ref, b_ref):
    xv = x_ref[...]
    a_ref[...] = _pack_bf16_pair(jnp.dot(xv, wa_ref[...], preferred_element_type=jnp.float32))
    b_ref[...] = _pack_bf16_pair(jnp.dot(xv, wb_ref[...], preferred_element_type=jnp.float32))


def _edge_body(gs_ref, gd_ref, e_ref, wc_ref, b1_ref, w2_ref, b2_ref, out_ref):
    ev = e_ref[...]
    gs_lo, gs_hi = _unpack_bf16_pair(gs_ref[...])
    gd_lo, gd_hi = _unpack_bf16_pair(gd_ref[...])
    h = jnp.concatenate([gs_lo + gd_lo, gs_hi + gd_hi], axis=-1) + b1_ref[...]
    h = h + jnp.dot(ev, wc_ref[...], preferred_element_type=jnp.float32)
    h = jnp.maximum(h, 0.0)
    out_ref[...] = jnp.dot(h, w2_ref[...], preferred_element_type=jnp.float32) + b2_ref[...] + ev


def _node_body(x_ref, g0_ref, g1_ref, wa_ref, wb_ref, b1_ref, w2_ref, b2_ref, out_ref):
    xv = x_ref[...]
    h = jnp.dot(xv, wa_ref[...], preferred_element_type=jnp.float32)
    h = h + jnp.dot(g0_ref[...] + g1_ref[...], wb_ref[...], preferred_element_type=jnp.float32)
    h = jnp.maximum(h + b1_ref[...], 0.0)
    out_ref[...] = jnp.dot(h, w2_ref[...], preferred_element_type=jnp.float32) + b2_ref[...] + xv


def _row_spec(blk):
    return pl.BlockSpec((blk, _D), lambda i: (i, 0))


def _half_spec(blk):
    return pl.BlockSpec((blk, _HALF), lambda i: (i, 0))


def _full_spec(shape):
    return pl.BlockSpec(shape, lambda i: (0,) * len(shape))


_proj = pl.pallas_call(
    _proj_body,
    grid=(_N_PAD // _BLK_N,),
    in_specs=[_row_spec(_BLK_N), _full_spec((_D, _D)), _full_spec((_D, _D))],
    out_specs=[_half_spec(_BLK_N), _half_spec(_BLK_N)],
    out_shape=[jax.ShapeDtypeStruct((_N_PAD, _HALF), jnp.int32)] * 2,
    compiler_params=pltpu.CompilerParams(dimension_semantics=("parallel",)),
)

_edge_mlp = pl.pallas_call(
    _edge_body,
    grid=(_EH // _BLK_E,),
    in_specs=[_half_spec(_BLK_E), _half_spec(_BLK_E), _row_spec(_BLK_E),
              _full_spec((_D, _D)), _full_spec((1, _D)),
              _full_spec((_D, _D)), _full_spec((1, _D))],
    out_specs=_row_spec(_BLK_E),
    out_shape=jax.ShapeDtypeStruct((_EH, _D), jnp.float32),
    compiler_params=pltpu.CompilerParams(dimension_semantics=("parallel",)),
)

_node_mlp = pl.pallas_call(
    _node_body,
    grid=(_N_PAD // _BLK_N,),
    in_specs=[_row_spec(_BLK_N), _row_spec(_BLK_N), _row_spec(_BLK_N),
              _full_spec((_D, _D)), _full_spec((_D, _D)), _full_spec((1, _D)),
              _full_spec((_D, _D)), _full_spec((1, _D))],
    out_specs=_row_spec(_BLK_N),
    out_shape=jax.ShapeDtypeStruct((_N_PAD, _D), jnp.float32),
    compiler_params=pltpu.CompilerParams(dimension_semantics=("parallel",)),
)


_SC_MESH = plsc.VectorSubcoreMesh(
    core_axis_name="c", subcore_axis_name="s", num_cores=_NC, num_subcores=_NS)


@functools.partial(
    pl.kernel,
    out_type=(jax.ShapeDtypeStruct((_EH, _HALF), jnp.int32),
              jax.ShapeDtypeStruct((_EH, _HALF), jnp.int32)),
    mesh=_SC_MESH,
    scratch_types=[
        pltpu.VMEM((_GCH, _CH), jnp.int32),
        pltpu.VMEM((_GCH, _CH), jnp.int32),
        pltpu.VMEM((_CH, _HALF), jnp.int32),
        pltpu.VMEM((_CH, _HALF), jnp.int32),
        pltpu.VMEM((_CH, _HALF), jnp.int32),
        pltpu.VMEM((_CH, _HALF), jnp.int32),
        pltpu.SemaphoreType.DMA,
        pltpu.SemaphoreType.DMA,
        pltpu.SemaphoreType.DMA,
        pltpu.SemaphoreType.DMA,
    ],
)
def _sc_gather(a_hbm, b_hbm, src_hbm, dst_hbm, gs_hbm, gd_hbm,
               srcv, dstv, bufa0, bufb0, bufa1, bufb1, sa0, sb0, sa1, sb1):
    wid = lax.axis_index("s") * _NC + lax.axis_index("c")
    row0 = wid * _GCH
    pltpu.sync_copy(src_hbm.at[wid], srcv)
    pltpu.sync_copy(dst_hbm.at[wid], dstv)

    def start(j, ba, bb, sa, sb):
        pltpu.async_copy(a_hbm.at[srcv.at[j]], ba, sa)
        pltpu.async_copy(b_hbm.at[dstv.at[j]], bb, sb)

    def finish(j, ba, bb, sa, sb):
        pltpu.make_async_copy(a_hbm.at[srcv.at[0]], ba, sa).wait()
        pltpu.make_async_copy(b_hbm.at[dstv.at[0]], bb, sb).wait()
        out0 = (row0 + j) * _CH
        pltpu.sync_copy(ba, gs_hbm.at[pl.ds(out0, _CH)])
        pltpu.sync_copy(bb, gd_hbm.at[pl.ds(out0, _CH)])

    start(0, bufa0, bufb0, sa0, sb0)

    def body(t, carry):
        j = 2 * t
        start(j + 1, bufa1, bufb1, sa1, sb1)
        finish(j, bufa0, bufb0, sa0, sb0)
        start(j + 2, bufa0, bufb0, sa0, sb0)
        finish(j + 1, bufa1, bufb1, sa1, sb1)
        return carry

    lax.fori_loop(0, _GCH // 2 - 1, body, 0)
    start(_GCH - 1, bufa1, bufb1, sa1, sb1)
    finish(_GCH - 2, bufa0, bufb0, sa0, sb0)
    finish(_GCH - 1, bufa1, bufb1, sa1, sb1)


@functools.partial(
    pl.kernel,
    out_type=jax.ShapeDtypeStruct((_N_PAD, _D), jnp.float32),
    mesh=_SC_MESH,
    scratch_types=[
        pltpu.VMEM((_SCH, _CH), jnp.int32),
        pltpu.VMEM((_CH, _HALF), jnp.float32),
        pltpu.VMEM((_CH, _HALF), jnp.float32),
        pltpu.VMEM_SHARED((_N_PAD, _HALF), jnp.float32),
        pltpu.SemaphoreType.DMA,
        pltpu.SemaphoreType.DMA,
    ],
)
def _sc_scatter(e_hbm, dst_hbm, agg_hbm, dstv, ebuf, ebuf1, acc, se0, se1):
    c = lax.axis_index("c")
    s = lax.axis_index("s")
    col0 = c * _HALF

    def zbody(r, carry):
        zero = jnp.zeros((16,), jnp.float32)
        for k in range(_HALF // 16):
            ebuf[r, pl.ds(k * 16, 16)] = zero
        return carry

    lax.fori_loop(0, _CH, zbody, 0)
    for k in range(_ROWS_T // _CH):
        pltpu.sync_copy(ebuf, acc.at[pl.ds(s * _ROWS_T + k * _CH, _CH)])
    plsc.subcore_barrier()

    row0 = s * _SCH
    pltpu.sync_copy(dst_hbm.at[pl.ds(row0, _SCH)], dstv)

    def start(j, buf, sem):
        e0 = (row0 + j) * _CH
        pltpu.async_copy(e_hbm.at[pl.ds(e0, _CH), pl.ds(col0, _HALF)], buf, sem)

    def finish(j, buf, sem):
        pltpu.make_async_copy(e_hbm.at[pl.ds(0, _CH), pl.ds(col0, _HALF)], buf, sem).wait()
        pltpu.sync_copy(buf, acc.at[dstv.at[j]], add=True)

    start(0, ebuf, se0)

    def body(t, carry):
        j = 2 * t
        start(j + 1, ebuf1, se1)
        finish(j, ebuf, se0)
        start(j + 2, ebuf, se0)
        finish(j + 1, ebuf1, se1)
        return carry

    lax.fori_loop(0, _SCH // 2 - 1, body, 0)
    start(_SCH - 1, ebuf1, se1)
    finish(_SCH - 2, ebuf, se0)
    finish(_SCH - 1, ebuf1, se1)
    plsc.subcore_barrier()

    pltpu.sync_copy(acc.at[pl.ds(s * _ROWS_T, _ROWS_T)],
                    agg_hbm.at[pl.ds(s * _ROWS_T, _ROWS_T), pl.ds(col0, _HALF)])


def kernel(x, edge_index, edge_attr, edge_W1, edge_b1, edge_W2, edge_b2,
           node_W1, node_b1, node_W2, node_b2):
    src = edge_index[0]
    dst = edge_index[1]
    pad_e = _E_PAD - _E
    src_flat = jnp.concatenate([src, jnp.zeros((pad_e,), jnp.int32)])
    dst_flat = jnp.concatenate([dst, jnp.full((pad_e,), _N, jnp.int32)])
    src_3d = src_flat.reshape(2, _NW, _GCH, _CH)
    dst_3d = dst_flat.reshape(2, _NW, _GCH, _CH)
    nrow_h = _EH // _CH
    dst_2d = dst_flat.reshape(-1, _CH)
    src_h = (src_3d[0], src_3d[1])
    dst_h = (dst_3d[0], dst_3d[1])
    dst_sh = (dst_2d[:nrow_h], dst_2d[nrow_h:])
    x_c = jnp.pad(x, ((0, _N_PAD - _N), (0, 0)))
    e_p = jnp.pad(edge_attr, ((0, pad_e), (0, 0)))
    e_h = [e_p[:_EH], e_p[_EH:]]

    for step in range(_STEPS):
        wa = edge_W1[step, :_D]
        wb = edge_W1[step, _D:2 * _D]
        wc = edge_W1[step, 2 * _D:]
        eb1 = edge_b1[step].reshape(1, _D)
        ew2 = edge_W2[step]
        eb2 = edge_b2[step].reshape(1, _D)
        na = node_W1[step, :_D]
        nb = node_W1[step, _D:]
        nb1 = node_b1[step].reshape(1, _D)
        nw2 = node_W2[step]
        nb2 = node_b2[step].reshape(1, _D)

        a_t, b_t = _proj(x_c, wa, wb)
        g = [_sc_gather(a_t, b_t, src_h[h], dst_h[h]) for h in range(2)]
        agg = [None, None]
        for h in range(2):
            e_h[h] = _edge_mlp(g[h][0], g[h][1], e_h[h], wc, eb1, ew2, eb2)
            agg[h] = _sc_scatter(e_h[h], dst_sh[h])
        x_c = _node_mlp(x_c, agg[0], agg[1], na, nb, nb1, nw2, nb2)

    return x_c[:_N], jnp.concatenate([e_h[0], e_h[1][:_E - _EH]])
